# baseline (speedup 1.0000x reference)
import functools

import jax
import jax.numpy as jnp
from jax import lax
from jax.experimental import pallas as pl
from jax.experimental.pallas import tpu as pltpu
from jax.experimental.pallas import tpu_sc as plsc

N_NODES = 10000
N_EDGES = 320000
NODE_DIM = 128
EDGE_DIM = 16
HIDDEN_DIM = 128

NC = 2
NS = 16
LANES = 16

EDGE_BLOCK_TC = 4000
NODE_BLOCK_TC = 1000
B = 40
NCHUNK = 5

E_CHUNK = N_EDGES // NCHUNK
CHUNK_STEPS = E_CHUNK // EDGE_BLOCK_TC
EPT_C = E_CHUNK // (NC * NS)
ROWS_PER_TILE = 624
ROWS_REM = N_NODES - NS * ROWS_PER_TILE


def _edge_mlp_body(ef_ref, w1_ref, b1_ref, w2_ref, b2_ref, ew_ref):
    h = jnp.dot(ef_ref[...], w1_ref[...], preferred_element_type=jnp.float32)
    h = h + b1_ref[...]
    h = h * jax.nn.sigmoid(h)
    ew = jnp.dot(h, w2_ref[...], preferred_element_type=jnp.float32)
    ew_ref[...] = ew + b2_ref[...]


def _edge_mlp_chunk(kc, ef, We1, be1, We2, be2):
    return pl.pallas_call(
        _edge_mlp_body,
        grid=(CHUNK_STEPS,),
        in_specs=[
            pl.BlockSpec((EDGE_BLOCK_TC, EDGE_DIM),
                         lambda i, kc=kc: (kc * CHUNK_STEPS + i, 0)),
            pl.BlockSpec((EDGE_DIM, HIDDEN_DIM), lambda i: (0, 0)),
            pl.BlockSpec((1, HIDDEN_DIM), lambda i: (0, 0)),
            pl.BlockSpec((HIDDEN_DIM, NODE_DIM), lambda i: (0, 0)),
            pl.BlockSpec((1, NODE_DIM), lambda i: (0, 0)),
        ],
        out_specs=pl.BlockSpec((EDGE_BLOCK_TC, NODE_DIM), lambda i: (i, 0)),
        out_shape=jax.ShapeDtypeStruct((E_CHUNK, NODE_DIM), jnp.float32),
    )(ef, We1, be1.reshape(1, -1), We2, be2.reshape(1, -1))


NB = EPT_C // B
NBUF = 4
NIB = 8


def _sc_body(kc, x_hbm, src_hbm, dst_hbm, ew_hbm, zero_hbm, out_hbm,
             acc, i0, i1, i2, i3, i4, i5, i6, i7,
             xr0, xr1, xr2, xr3, ew0, ew1, ew2, ew3,
             is0, is1, is2, is3, is4, is5, is6, is7,
             g0, g1, g2, g3,
             e0s, e1s, e2s, e3s, s0, s1, s2, s3):
    c = lax.axis_index("c")
    s = lax.axis_index("s")
    ib = (i0, i1, i2, i3, i4, i5, i6, i7)
    xr = (xr0, xr1, xr2, xr3)
    ewr = (ew0, ew1, ew2, ew3)
    isem = (is0, is1, is2, is3, is4, is5, is6, is7)
    gsem = (g0, g1, g2, g3)
    esem = (e0s, e1s, e2s, e3s)
    ssem = (s0, s1, s2, s3)

    r0 = pl.multiple_of(s * ROWS_PER_TILE, 8)
    pltpu.sync_copy(zero_hbm.at[pl.ds(r0, ROWS_PER_TILE)],
                    acc.at[pl.ds(r0, ROWS_PER_TILE)])

    @pl.when(s == NS - 1)
    def _zero_rem():
        rr = NS * ROWS_PER_TILE
        pltpu.sync_copy(zero_hbm.at[pl.ds(rr, ROWS_REM)],
                        acc.at[pl.ds(rr, ROWS_REM)])

    plsc.subcore_barrier()

    w = c * NS + s
    ebase = w * EPT_C

    def load_idx(b, si):
        pltpu.async_copy(src_hbm.at[w].at[b], ib[si].at[0], isem[si])
        pltpu.async_copy(dst_hbm.at[w].at[b], ib[si].at[1], isem[si])

    def wait_idx(b, si):
        pltpu.make_async_copy(src_hbm.at[w].at[b], ib[si].at[0],
                              isem[si]).wait()
        pltpu.make_async_copy(dst_hbm.at[w].at[b], ib[si].at[1],
                              isem[si]).wait()

    def load_data(b, si, sd):
        eoff = pl.multiple_of(ebase + b * B, 8)
        pltpu.async_copy(x_hbm.at[ib[si].at[0]], xr[sd], gsem[sd])
        pltpu.async_copy(ew_hbm.at[pl.ds(eoff, B)], ewr[sd], esem[sd])

    def wait_data(b, si, sd):
        pltpu.make_async_copy(x_hbm.at[ib[si].at[0]], xr[sd],
                              gsem[sd]).wait()
        eoff = pl.multiple_of(ebase + b * B, 8)
        pltpu.make_async_copy(ew_hbm.at[pl.ds(eoff, B)], ewr[sd],
                              esem[sd]).wait()

    def mul(sd):
        @pl.loop(0, B, step=2)
        def _row(r):
            for dr in range(2):
                for k in range(0, NODE_DIM, LANES):
                    sl = (r + dr, pl.ds(k, LANES))
                    xr[sd].at[*sl][...] = (xr[sd].at[*sl][...]
                                           * ewr[sd].at[*sl][...])

    def scatter(si, sd):
        pltpu.async_copy(xr[sd], acc.at[ib[si].at[1]], ssem[sd], add=True)

    def wait_scatter(si, sd):
        pltpu.make_async_copy(xr[sd], acc.at[ib[si].at[1]], ssem[sd]).wait()

    for b in (0, 1, 2):
        load_idx(b, b % NIB)
    for b in (0, 1):
        wait_idx(b, b % NIB)
        load_data(b, b % NIB, b % NBUF)

    def stage(b, st):
        inb = b + 3
        nb = b + 2

        @pl.when(inb < NB)
        def _pf_idx():
            load_idx(inb, (st + 3) % NIB)

        @pl.when(nb < NB)
        def _pf_data():
            @pl.when(nb >= NBUF)
            def _drain():
                wait_scatter((st - 2) % NIB, (st - 2) % NBUF)

            wait_idx(nb, (st + 2) % NIB)
            load_data(nb, (st + 2) % NIB, (st + 2) % NBUF)

        @pl.when(b < NB)
        def _work():
            wait_data(b, st % NIB, st % NBUF)
            mul(st % NBUF)
            scatter(st % NIB, st % NBUF)

    n_groups = (NB + NIB - 1) // NIB

    @pl.loop(0, n_groups)
    def _grp(k):
        kb = k * NIB
        for st in range(NIB):
            stage(kb + st, st)

    for b in range(NB - NBUF, NB):
        wait_scatter(b % NIB, b % NBUF)

    plsc.subcore_barrier()
    pltpu.sync_copy(acc.at[pl.ds(r0, ROWS_PER_TILE)],
                    out_hbm.at[c].at[pl.ds(r0, ROWS_PER_TILE)])

    @pl.when(s == NS - 1)
    def _out_rem():
        rr = NS * ROWS_PER_TILE
        pltpu.sync_copy(acc.at[pl.ds(rr, ROWS_REM)],
                        out_hbm.at[c].at[pl.ds(rr, ROWS_REM)])


def _gather_mul_scatter(kc, x, srck, dstk, ew, zeros):
    mesh = plsc.VectorSubcoreMesh(core_axis_name="c", subcore_axis_name="s")
    dma = pltpu.SemaphoreType.DMA
    fn = pl.kernel(
        functools.partial(_sc_body, kc),
        out_type=jax.ShapeDtypeStruct((NC, N_NODES, NODE_DIM), jnp.float32),
        mesh=mesh,
        scratch_types=(
            [pltpu.VMEM_SHARED((N_NODES, NODE_DIM), jnp.float32)]
            + [pltpu.VMEM((2, B), jnp.int32) for _ in range(NIB)]
            + [pltpu.VMEM((B, NODE_DIM), jnp.float32) for _ in range(2 * NBUF)]
            + [dma for _ in range(NIB + 3 * NBUF)]
        ),
    )
    return fn(x, srck, dstk, ew, zeros)


def _node_mlp_body(*refs):
    p_refs = refs[:NCHUNK]
    x_ref, w1_ref, b1_ref, w2_ref, b2_ref, y_ref = refs[NCHUNK:]
    agg = p_refs[0][0] + p_refs[0][1]
    for p in p_refs[1:]:
        agg = agg + p[0] + p[1]
    g = jnp.dot(agg, w1_ref[...], preferred_element_type=jnp.float32)
    g = g + b1_ref[...]
    g = g * jax.nn.sigmoid(g)
    o = jnp.dot(g, w2_ref[...], preferred_element_type=jnp.float32)
    y_ref[...] = x_ref[...] + o + b2_ref[...]


def _node_mlp(partials, x, Wn1, bn1, Wn2, bn2):
    grid = (N_NODES // NODE_BLOCK_TC,)
    return pl.pallas_call(
        _node_mlp_body,
        grid=grid,
        in_specs=(
            [pl.BlockSpec((NC, NODE_BLOCK_TC, NODE_DIM), lambda i: (0, i, 0))
             for _ in range(NCHUNK)]
            + [
                pl.BlockSpec((NODE_BLOCK_TC, NODE_DIM), lambda i: (i, 0)),
                pl.BlockSpec((NODE_DIM, HIDDEN_DIM), lambda i: (0, 0)),
                pl.BlockSpec((1, HIDDEN_DIM), lambda i: (0, 0)),
                pl.BlockSpec((HIDDEN_DIM, NODE_DIM), lambda i: (0, 0)),
                pl.BlockSpec((1, NODE_DIM), lambda i: (0, 0)),
            ]
        ),
        out_specs=pl.BlockSpec((NODE_BLOCK_TC, NODE_DIM), lambda i: (i, 0)),
        out_shape=jax.ShapeDtypeStruct((N_NODES, NODE_DIM), jnp.float32),
    )(*partials, x, Wn1, bn1.reshape(1, -1), Wn2, bn2.reshape(1, -1))


def kernel(x, edge_index, edge_features, We1, be1, We2, be2, Wn1, bn1, Wn2, bn2):
    src = edge_index[0].astype(jnp.int32).reshape(NCHUNK, NC * NS, NB, B)
    dst = edge_index[1].astype(jnp.int32).reshape(NCHUNK, NC * NS, NB, B)
    zeros = jnp.zeros((N_NODES, NODE_DIM), jnp.float32)
    partials = []
    for kc in range(NCHUNK):
        ew = _edge_mlp_chunk(kc, edge_features, We1, be1, We2, be2)
        partials.append(_gather_mul_scatter(kc, x, src[kc], dst[kc], ew,
                                            zeros))
    return _node_mlp(partials, x, Wn1, bn1, Wn2, bn2)


# speedup vs baseline: 1.0140x; 1.0140x over previous
import functools

import jax
import jax.numpy as jnp
from jax import lax
from jax.experimental import pallas as pl
from jax.experimental.pallas import tpu as pltpu
from jax.experimental.pallas import tpu_sc as plsc

N_NODES = 10000
N_EDGES = 320000
NODE_DIM = 128
EDGE_DIM = 16
HIDDEN_DIM = 128

NC = 2
NS = 16
LANES = 16

NODE_BLOCK_TC = 1000
B = 40
NCHUNK = 5

E_PER_TILE = N_EDGES // (NC * NS)
EPT_C = E_PER_TILE // NCHUNK
E_CHUNK = N_EDGES // NCHUNK
EDGE_BLOCK_TC = EPT_C
NBT = N_EDGES // B // (NC * NS)
ROWS_PER_TILE = 624
ROWS_REM = N_NODES - NS * ROWS_PER_TILE


def _edge_mlp_body(ef_ref, w1_ref, b1_ref, w2_ref, b2_ref, ew_ref):
    h = jnp.dot(ef_ref[...], w1_ref[...], preferred_element_type=jnp.float32)
    h = h + b1_ref[...]
    h = h * jax.nn.sigmoid(h)
    ew = jnp.dot(h, w2_ref[...], preferred_element_type=jnp.float32)
    ew_ref[...] = ew + b2_ref[...]


def _edge_mlp_chunk(kc, ef, We1, be1, We2, be2):
    return pl.pallas_call(
        _edge_mlp_body,
        grid=(NC * NS,),
        in_specs=[
            pl.BlockSpec((EDGE_BLOCK_TC, EDGE_DIM),
                         lambda i, kc=kc: (i * NCHUNK + kc, 0)),
            pl.BlockSpec((EDGE_DIM, HIDDEN_DIM), lambda i: (0, 0)),
            pl.BlockSpec((1, HIDDEN_DIM), lambda i: (0, 0)),
            pl.BlockSpec((HIDDEN_DIM, NODE_DIM), lambda i: (0, 0)),
            pl.BlockSpec((1, NODE_DIM), lambda i: (0, 0)),
        ],
        out_specs=pl.BlockSpec((EDGE_BLOCK_TC, NODE_DIM), lambda i: (i, 0)),
        out_shape=jax.ShapeDtypeStruct((E_CHUNK, NODE_DIM), jnp.float32),
    )(ef, We1, be1.reshape(1, -1), We2, be2.reshape(1, -1))


NB = EPT_C // B
NBUF = 4
NIB = 8


def _sc_body(kc, x_hbm, src_hbm, dst_hbm, ew_hbm, zero_hbm, out_hbm,
             acc, i0, i1, i2, i3, i4, i5, i6, i7,
             xr0, xr1, xr2, xr3, ew0, ew1, ew2, ew3,
             is0, is1, is2, is3, is4, is5, is6, is7,
             g0, g1, g2, g3,
             e0s, e1s, e2s, e3s, s0, s1, s2, s3):
    c = lax.axis_index("c")
    s = lax.axis_index("s")
    ib = (i0, i1, i2, i3, i4, i5, i6, i7)
    xr = (xr0, xr1, xr2, xr3)
    ewr = (ew0, ew1, ew2, ew3)
    isem = (is0, is1, is2, is3, is4, is5, is6, is7)
    gsem = (g0, g1, g2, g3)
    esem = (e0s, e1s, e2s, e3s)
    ssem = (s0, s1, s2, s3)

    r0 = pl.multiple_of(s * ROWS_PER_TILE, 8)
    pltpu.sync_copy(zero_hbm.at[pl.ds(r0, ROWS_PER_TILE)],
                    acc.at[pl.ds(r0, ROWS_PER_TILE)])

    @pl.when(s == NS - 1)
    def _zero_rem():
        rr = NS * ROWS_PER_TILE
        pltpu.sync_copy(zero_hbm.at[pl.ds(rr, ROWS_REM)],
                        acc.at[pl.ds(rr, ROWS_REM)])

    plsc.subcore_barrier()

    w = c * NS + s
    ebase = w * EPT_C

    boff = kc * NB

    def load_idx(b, si):
        pltpu.async_copy(src_hbm.at[w].at[boff + b], ib[si].at[0], isem[si])
        pltpu.async_copy(dst_hbm.at[w].at[boff + b], ib[si].at[1], isem[si])

    def wait_idx(b, si):
        pltpu.make_async_copy(src_hbm.at[w].at[boff + b], ib[si].at[0],
                              isem[si]).wait()
        pltpu.make_async_copy(dst_hbm.at[w].at[boff + b], ib[si].at[1],
                              isem[si]).wait()

    def load_data(b, si, sd):
        eoff = pl.multiple_of(ebase + b * B, 8)
        pltpu.async_copy(x_hbm.at[ib[si].at[0]], xr[sd], gsem[sd])
        pltpu.async_copy(ew_hbm.at[pl.ds(eoff, B)], ewr[sd], esem[sd])

    def wait_data(b, si, sd):
        pltpu.make_async_copy(x_hbm.at[ib[si].at[0]], xr[sd],
                              gsem[sd]).wait()
        eoff = pl.multiple_of(ebase + b * B, 8)
        pltpu.make_async_copy(ew_hbm.at[pl.ds(eoff, B)], ewr[sd],
                              esem[sd]).wait()

    def mul(sd):
        @pl.loop(0, B, step=2)
        def _row(r):
            for dr in range(2):
                for k in range(0, NODE_DIM, LANES):
                    sl = (r + dr, pl.ds(k, LANES))
                    xr[sd].at[*sl][...] = (xr[sd].at[*sl][...]
                                           * ewr[sd].at[*sl][...])

    def scatter(si, sd):
        pltpu.async_copy(xr[sd], acc.at[ib[si].at[1]], ssem[sd], add=True)

    def wait_scatter(si, sd):
        pltpu.make_async_copy(xr[sd], acc.at[ib[si].at[1]], ssem[sd]).wait()

    for b in (0, 1, 2):
        load_idx(b, b % NIB)
    for b in (0, 1):
        wait_idx(b, b % NIB)
        load_data(b, b % NIB, b % NBUF)

    def stage(b, st):
        inb = b + 3
        nb = b + 2

        @pl.when(inb < NB)
        def _pf_idx():
            load_idx(inb, (st + 3) % NIB)

        @pl.when(nb < NB)
        def _pf_data():
            @pl.when(nb >= NBUF)
            def _drain():
                wait_scatter((st - 2) % NIB, (st - 2) % NBUF)

            wait_idx(nb, (st + 2) % NIB)
            load_data(nb, (st + 2) % NIB, (st + 2) % NBUF)

        @pl.when(b < NB)
        def _work():
            wait_data(b, st % NIB, st % NBUF)
            mul(st % NBUF)
            scatter(st % NIB, st % NBUF)

    n_groups = (NB + NIB - 1) // NIB

    @pl.loop(0, n_groups)
    def _grp(k):
        kb = k * NIB
        for st in range(NIB):
            stage(kb + st, st)

    for b in range(NB - NBUF, NB):
        wait_scatter(b % NIB, b % NBUF)

    plsc.subcore_barrier()
    pltpu.sync_copy(acc.at[pl.ds(r0, ROWS_PER_TILE)],
                    out_hbm.at[c].at[pl.ds(r0, ROWS_PER_TILE)])

    @pl.when(s == NS - 1)
    def _out_rem():
        rr = NS * ROWS_PER_TILE
        pltpu.sync_copy(acc.at[pl.ds(rr, ROWS_REM)],
                        out_hbm.at[c].at[pl.ds(rr, ROWS_REM)])


def _gather_mul_scatter(kc, x, srck, dstk, ew, zeros):
    mesh = plsc.VectorSubcoreMesh(core_axis_name="c", subcore_axis_name="s")
    dma = pltpu.SemaphoreType.DMA
    fn = pl.kernel(
        functools.partial(_sc_body, kc),
        out_type=jax.ShapeDtypeStruct((NC, N_NODES, NODE_DIM), jnp.float32),
        mesh=mesh,
        scratch_types=(
            [pltpu.VMEM_SHARED((N_NODES, NODE_DIM), jnp.float32)]
            + [pltpu.VMEM((2, B), jnp.int32) for _ in range(NIB)]
            + [pltpu.VMEM((B, NODE_DIM), jnp.float32) for _ in range(2 * NBUF)]
            + [dma for _ in range(NIB + 3 * NBUF)]
        ),
    )
    return fn(x, srck, dstk, ew, zeros)


def _node_mlp_body(*refs):
    p_refs = refs[:NCHUNK]
    x_ref, w1_ref, b1_ref, w2_ref, b2_ref, y_ref = refs[NCHUNK:]
    agg = p_refs[0][0] + p_refs[0][1]
    for p in p_refs[1:]:
        agg = agg + p[0] + p[1]
    g = jnp.dot(agg, w1_ref[...], preferred_element_type=jnp.float32)
    g = g + b1_ref[...]
    g = g * jax.nn.sigmoid(g)
    o = jnp.dot(g, w2_ref[...], preferred_element_type=jnp.float32)
    y_ref[...] = x_ref[...] + o + b2_ref[...]


def _node_mlp(partials, x, Wn1, bn1, Wn2, bn2):
    grid = (N_NODES // NODE_BLOCK_TC,)
    return pl.pallas_call(
        _node_mlp_body,
        grid=grid,
        in_specs=(
            [pl.BlockSpec((NC, NODE_BLOCK_TC, NODE_DIM), lambda i: (0, i, 0))
             for _ in range(NCHUNK)]
            + [
                pl.BlockSpec((NODE_BLOCK_TC, NODE_DIM), lambda i: (i, 0)),
                pl.BlockSpec((NODE_DIM, HIDDEN_DIM), lambda i: (0, 0)),
                pl.BlockSpec((1, HIDDEN_DIM), lambda i: (0, 0)),
                pl.BlockSpec((HIDDEN_DIM, NODE_DIM), lambda i: (0, 0)),
                pl.BlockSpec((1, NODE_DIM), lambda i: (0, 0)),
            ]
        ),
        out_specs=pl.BlockSpec((NODE_BLOCK_TC, NODE_DIM), lambda i: (i, 0)),
        out_shape=jax.ShapeDtypeStruct((N_NODES, NODE_DIM), jnp.float32),
    )(*partials, x, Wn1, bn1.reshape(1, -1), Wn2, bn2.reshape(1, -1))


def kernel(x, edge_index, edge_features, We1, be1, We2, be2, Wn1, bn1, Wn2, bn2):
    src = edge_index[0].astype(jnp.int32).reshape(NC * NS, NBT, B)
    dst = edge_index[1].astype(jnp.int32).reshape(NC * NS, NBT, B)
    zeros = jnp.zeros((N_NODES, NODE_DIM), jnp.float32)
    partials = []
    for kc in range(NCHUNK):
        ew = _edge_mlp_chunk(kc, edge_features, We1, be1, We2, be2)
        partials.append(_gather_mul_scatter(kc, x, src, dst, ew, zeros))
    return _node_mlp(partials, x, Wn1, bn1, Wn2, bn2)


# speedup vs baseline: 1.1378x; 1.1221x over previous
import functools

import jax
import jax.numpy as jnp
from jax import lax
from jax.experimental import pallas as pl
from jax.experimental.pallas import tpu as pltpu
from jax.experimental.pallas import tpu_sc as plsc

N_NODES = 10000
N_EDGES = 320000
NODE_DIM = 128
EDGE_DIM = 16
HIDDEN_DIM = 128

NC = 2
NS = 16
LANES = 16

NODE_BLOCK_TC = 1000
B = 40
NCHUNK = 2

E_PER_TILE = N_EDGES // (NC * NS)
EPT_C = E_PER_TILE // NCHUNK
E_CHUNK = N_EDGES // NCHUNK
EDGE_BLOCK_TC = EPT_C
ROWS_PER_TILE = 624
ROWS_REM = N_NODES - NS * ROWS_PER_TILE


def _edge_mlp_body(ef_ref, w1_ref, b1_ref, w2_ref, b2_ref, ew_ref):
    h = jnp.dot(ef_ref[...], w1_ref[...], preferred_element_type=jnp.float32)
    h = h + b1_ref[...]
    h = h * jax.nn.sigmoid(h)
    ew = jnp.dot(h, w2_ref[...], preferred_element_type=jnp.float32)
    ew_ref[...] = ew + b2_ref[...]


def _edge_mlp_chunk(kc, ef, We1, be1, We2, be2):
    return pl.pallas_call(
        _edge_mlp_body,
        grid=(NC * NS,),
        in_specs=[
            pl.BlockSpec((EDGE_BLOCK_TC, EDGE_DIM),
                         lambda i, kc=kc: (i * NCHUNK + kc, 0)),
            pl.BlockSpec((EDGE_DIM, HIDDEN_DIM), lambda i: (0, 0)),
            pl.BlockSpec((1, HIDDEN_DIM), lambda i: (0, 0)),
            pl.BlockSpec((HIDDEN_DIM, NODE_DIM), lambda i: (0, 0)),
            pl.BlockSpec((1, NODE_DIM), lambda i: (0, 0)),
        ],
        out_specs=pl.BlockSpec((EDGE_BLOCK_TC, NODE_DIM), lambda i: (i, 0)),
        out_shape=jax.ShapeDtypeStruct((E_CHUNK, NODE_DIM), jnp.float32),
    )(ef, We1, be1.reshape(1, -1), We2, be2.reshape(1, -1))


NB = EPT_C // B
NBUF = 4
NIB = 8


def _sc_body(kc, x_hbm, src_hbm, dst_hbm, ew_hbm, zero_hbm, out_hbm,
             acc, i0, i1, i2, i3, i4, i5, i6, i7,
             xr0, xr1, xr2, xr3, ew0, ew1, ew2, ew3,
             is0, is1, is2, is3, is4, is5, is6, is7,
             g0, g1, g2, g3,
             e0s, e1s, e2s, e3s, s0, s1, s2, s3):
    c = lax.axis_index("c")
    s = lax.axis_index("s")
    ib = (i0, i1, i2, i3, i4, i5, i6, i7)
    xr = (xr0, xr1, xr2, xr3)
    ewr = (ew0, ew1, ew2, ew3)
    isem = (is0, is1, is2, is3, is4, is5, is6, is7)
    gsem = (g0, g1, g2, g3)
    esem = (e0s, e1s, e2s, e3s)
    ssem = (s0, s1, s2, s3)

    r0 = pl.multiple_of(s * ROWS_PER_TILE, 8)
    pltpu.sync_copy(zero_hbm.at[pl.ds(r0, ROWS_PER_TILE)],
                    acc.at[pl.ds(r0, ROWS_PER_TILE)])

    @pl.when(s == NS - 1)
    def _zero_rem():
        rr = NS * ROWS_PER_TILE
        pltpu.sync_copy(zero_hbm.at[pl.ds(rr, ROWS_REM)],
                        acc.at[pl.ds(rr, ROWS_REM)])

    plsc.subcore_barrier()

    w = c * NS + s
    ebase = w * EPT_C

    ibase = w * E_PER_TILE + kc * EPT_C

    def load_idx(b, si):
        ioff = pl.multiple_of(ibase + b * B, 8)
        pltpu.async_copy(src_hbm.at[pl.ds(ioff, B)], ib[si].at[0], isem[si])
        pltpu.async_copy(dst_hbm.at[pl.ds(ioff, B)], ib[si].at[1], isem[si])

    def wait_idx(b, si):
        ioff = pl.multiple_of(ibase + b * B, 8)
        pltpu.make_async_copy(src_hbm.at[pl.ds(ioff, B)], ib[si].at[0],
                              isem[si]).wait()
        pltpu.make_async_copy(dst_hbm.at[pl.ds(ioff, B)], ib[si].at[1],
                              isem[si]).wait()

    def load_data(b, si, sd):
        eoff = pl.multiple_of(ebase + b * B, 8)
        pltpu.async_copy(x_hbm.at[ib[si].at[0]], xr[sd], gsem[sd])
        pltpu.async_copy(ew_hbm.at[pl.ds(eoff, B)], ewr[sd], esem[sd])

    def wait_data(b, si, sd):
        pltpu.make_async_copy(x_hbm.at[ib[si].at[0]], xr[sd],
                              gsem[sd]).wait()
        eoff = pl.multiple_of(ebase + b * B, 8)
        pltpu.make_async_copy(ew_hbm.at[pl.ds(eoff, B)], ewr[sd],
                              esem[sd]).wait()

    def mul(sd):
        @pl.loop(0, B, step=2)
        def _row(r):
            for dr in range(2):
                for k in range(0, NODE_DIM, LANES):
                    sl = (r + dr, pl.ds(k, LANES))
                    xr[sd].at[*sl][...] = (xr[sd].at[*sl][...]
                                           * ewr[sd].at[*sl][...])

    def scatter(si, sd):
        pltpu.async_copy(xr[sd], acc.at[ib[si].at[1]], ssem[sd], add=True)

    def wait_scatter(si, sd):
        pltpu.make_async_copy(xr[sd], acc.at[ib[si].at[1]], ssem[sd]).wait()

    for b in (0, 1, 2):
        load_idx(b, b % NIB)
    for b in (0, 1):
        wait_idx(b, b % NIB)
        load_data(b, b % NIB, b % NBUF)

    def stage(b, st):
        inb = b + 3
        nb = b + 2

        @pl.when(inb < NB)
        def _pf_idx():
            load_idx(inb, (st + 3) % NIB)

        @pl.when(nb < NB)
        def _pf_data():
            @pl.when(nb >= NBUF)
            def _drain():
                wait_scatter((st - 2) % NIB, (st - 2) % NBUF)

            wait_idx(nb, (st + 2) % NIB)
            load_data(nb, (st + 2) % NIB, (st + 2) % NBUF)

        @pl.when(b < NB)
        def _work():
            wait_data(b, st % NIB, st % NBUF)
            mul(st % NBUF)
            scatter(st % NIB, st % NBUF)

    n_groups = (NB + NIB - 1) // NIB

    @pl.loop(0, n_groups)
    def _grp(k):
        kb = k * NIB
        for st in range(NIB):
            stage(kb + st, st)

    for b in range(NB - NBUF, NB):
        wait_scatter(b % NIB, b % NBUF)

    plsc.subcore_barrier()
    pltpu.sync_copy(acc.at[pl.ds(r0, ROWS_PER_TILE)],
                    out_hbm.at[c].at[pl.ds(r0, ROWS_PER_TILE)])

    @pl.when(s == NS - 1)
    def _out_rem():
        rr = NS * ROWS_PER_TILE
        pltpu.sync_copy(acc.at[pl.ds(rr, ROWS_REM)],
                        out_hbm.at[c].at[pl.ds(rr, ROWS_REM)])


def _gather_mul_scatter(kc, x, srck, dstk, ew, zeros):
    mesh = plsc.VectorSubcoreMesh(core_axis_name="c", subcore_axis_name="s")
    dma = pltpu.SemaphoreType.DMA
    fn = pl.kernel(
        functools.partial(_sc_body, kc),
        out_type=jax.ShapeDtypeStruct((NC, N_NODES, NODE_DIM), jnp.float32),
        mesh=mesh,
        scratch_types=(
            [pltpu.VMEM_SHARED((N_NODES, NODE_DIM), jnp.float32)]
            + [pltpu.VMEM((2, B), jnp.int32) for _ in range(NIB)]
            + [pltpu.VMEM((B, NODE_DIM), jnp.float32) for _ in range(2 * NBUF)]
            + [dma for _ in range(NIB + 3 * NBUF)]
        ),
    )
    return fn(x, srck, dstk, ew, zeros)


def _node_mlp_body(*refs):
    p_refs = refs[:NCHUNK]
    x_ref, w1_ref, b1_ref, w2_ref, b2_ref, y_ref = refs[NCHUNK:]
    agg = p_refs[0][0] + p_refs[0][1]
    for p in p_refs[1:]:
        agg = agg + p[0] + p[1]
    g = jnp.dot(agg, w1_ref[...], preferred_element_type=jnp.float32)
    g = g + b1_ref[...]
    g = g * jax.nn.sigmoid(g)
    o = jnp.dot(g, w2_ref[...], preferred_element_type=jnp.float32)
    y_ref[...] = x_ref[...] + o + b2_ref[...]


def _node_mlp(partials, x, Wn1, bn1, Wn2, bn2):
    grid = (N_NODES // NODE_BLOCK_TC,)
    return pl.pallas_call(
        _node_mlp_body,
        grid=grid,
        in_specs=(
            [pl.BlockSpec((NC, NODE_BLOCK_TC, NODE_DIM), lambda i: (0, i, 0))
             for _ in range(NCHUNK)]
            + [
                pl.BlockSpec((NODE_BLOCK_TC, NODE_DIM), lambda i: (i, 0)),
                pl.BlockSpec((NODE_DIM, HIDDEN_DIM), lambda i: (0, 0)),
                pl.BlockSpec((1, HIDDEN_DIM), lambda i: (0, 0)),
                pl.BlockSpec((HIDDEN_DIM, NODE_DIM), lambda i: (0, 0)),
                pl.BlockSpec((1, NODE_DIM), lambda i: (0, 0)),
            ]
        ),
        out_specs=pl.BlockSpec((NODE_BLOCK_TC, NODE_DIM), lambda i: (i, 0)),
        out_shape=jax.ShapeDtypeStruct((N_NODES, NODE_DIM), jnp.float32),
    )(*partials, x, Wn1, bn1.reshape(1, -1), Wn2, bn2.reshape(1, -1))


def kernel(x, edge_index, edge_features, We1, be1, We2, be2, Wn1, bn1, Wn2, bn2):
    src = edge_index[0].astype(jnp.int32)
    dst = edge_index[1].astype(jnp.int32)
    zeros = jnp.zeros((N_NODES, NODE_DIM), jnp.float32)
    partials = []
    for kc in range(NCHUNK):
        ew = _edge_mlp_chunk(kc, edge_features, We1, be1, We2, be2)
        partials.append(_gather_mul_scatter(kc, x, src, dst, ew, zeros))
    return _node_mlp(partials, x, Wn1, bn1, Wn2, bn2)
